# baseline (speedup 1.0000x reference)
import functools

import jax
import jax.numpy as jnp
from jax import lax
from jax.experimental import pallas as pl
from jax.experimental.pallas import tpu as pltpu
from jax.experimental.pallas import tpu_sc as plsc

_D_EMBED = 64
_SCALE = float(_D_EMBED) ** 0.5
_LANES = 16
_NUM_WORKERS = 32
_CHUNK = 512


def _make_lookup(batch: int):
    assert batch % (_NUM_WORKERS * _CHUNK) == 0
    bpw = batch // _NUM_WORKERS
    nchunk = bpw // _CHUNK
    mesh = plsc.VectorSubcoreMesh(core_axis_name="c", subcore_axis_name="s")

    @functools.partial(
        pl.kernel,
        mesh=mesh,
        out_type=jax.ShapeDtypeStruct((batch, _D_EMBED), jnp.float32),
        scratch_types=[
            pltpu.VMEM((bpw,), jnp.int32),
            pltpu.VMEM((_CHUNK, _D_EMBED), jnp.float32),
            pltpu.SemaphoreType.DMA,
        ],
        compiler_params=pltpu.CompilerParams(use_tc_tiling_on_sc=False),
    )
    def lookup(table_hbm, idx_hbm, out_hbm, idx_v, rows_v, sem):
        wid = lax.axis_index("s") * 2 + lax.axis_index("c")
        base = wid * bpw
        pltpu.sync_copy(idx_hbm.at[pl.ds(base, bpw)], idx_v)

        def chunk_body(g, _):
            pltpu.async_copy(
                table_hbm.at[idx_v.at[pl.ds(g * _CHUNK, _CHUNK)]], rows_v, sem
            ).wait()

            def scale_body(i, _):
                for j in range(_D_EMBED // _LANES):
                    sl = pl.ds(j * _LANES, _LANES)
                    rows_v[i, sl] = rows_v[i, sl] * _SCALE
                return 0

            lax.fori_loop(0, _CHUNK, scale_body, 0)
            pltpu.sync_copy(rows_v, out_hbm.at[pl.ds(base + g * _CHUNK, _CHUNK)])
            return 0

        lax.fori_loop(0, nchunk, chunk_body, 0)

    return lookup


def kernel(inp, emb_weight):
    b, t = inp.shape
    flat_idx = inp.reshape(b * t)
    out = _make_lookup(b * t)(emb_weight, flat_idx)
    return out.reshape(b, t, _D_EMBED)


# speedup vs baseline: 1.1156x; 1.1156x over previous
import functools

import jax
import jax.numpy as jnp
from jax import lax
from jax.experimental import pallas as pl
from jax.experimental.pallas import tpu as pltpu
from jax.experimental.pallas import tpu_sc as plsc

_D_EMBED = 64
_SCALE = float(_D_EMBED) ** 0.5
_LANES = 16
_NUM_WORKERS = 32
_CHUNK = 256
_NBUF = 2


def _make_lookup(batch: int):
    assert batch % (_NUM_WORKERS * _CHUNK * _NBUF) == 0
    bpw = batch // _NUM_WORKERS
    nchunk = bpw // _CHUNK
    mesh = plsc.VectorSubcoreMesh(core_axis_name="c", subcore_axis_name="s")

    @functools.partial(
        pl.kernel,
        mesh=mesh,
        out_type=jax.ShapeDtypeStruct((batch, _D_EMBED), jnp.float32),
        scratch_types=[
            pltpu.VMEM((bpw,), jnp.int32),
            pltpu.VMEM((_NBUF, _CHUNK, _D_EMBED), jnp.float32),
            pltpu.VMEM((_NBUF, _CHUNK, _D_EMBED), jnp.float32),
            [pltpu.SemaphoreType.DMA] * _NBUF,
            [pltpu.SemaphoreType.DMA] * _NBUF,
        ],
        compiler_params=pltpu.CompilerParams(use_tc_tiling_on_sc=False),
    )
    def lookup(table_hbm, idx_hbm, out_hbm, idx_v, gbuf, wbuf, gsems, wsems):
        wid = lax.axis_index("s") * 2 + lax.axis_index("c")
        base = wid * bpw
        pltpu.sync_copy(idx_hbm.at[pl.ds(base, bpw)], idx_v)

        def gather_start(chunk, b):
            pltpu.async_copy(
                table_hbm.at[idx_v.at[pl.ds(chunk * _CHUNK, _CHUNK)]],
                gbuf.at[b],
                gsems[b],
            )

        for b in range(_NBUF):
            gather_start(b, b)

        @pl.loop(0, nchunk, step=_NBUF)
        def _(g0):
            for b in range(_NBUF):
                g = g0 + b

                @pl.when(g >= _NBUF)
                def _():
                    pltpu.make_async_copy(
                        wbuf.at[b], out_hbm.at[pl.ds(base, _CHUNK)], wsems[b]
                    ).wait()

                pltpu.make_async_copy(
                    table_hbm.at[idx_v.at[pl.ds(g * _CHUNK, _CHUNK)]],
                    gbuf.at[b],
                    gsems[b],
                ).wait()

                @plsc.parallel_loop(0, _CHUNK, unroll=8)
                def _(i):
                    for j in range(_D_EMBED // _LANES):
                        sl = pl.ds(j * _LANES, _LANES)
                        wbuf[b, i, sl] = gbuf[b, i, sl] * _SCALE

                pltpu.async_copy(
                    wbuf.at[b], out_hbm.at[pl.ds(base + g * _CHUNK, _CHUNK)], wsems[b]
                )

                @pl.when(g + _NBUF < nchunk)
                def _():
                    gather_start(g + _NBUF, b)

        for b in range(_NBUF):
            pltpu.make_async_copy(
                wbuf.at[b], out_hbm.at[pl.ds(base, _CHUNK)], wsems[b]
            ).wait()

    return lookup


def kernel(inp, emb_weight):
    b, t = inp.shape
    flat_idx = inp.reshape(b * t)
    out = _make_lookup(b * t)(emb_weight, flat_idx)
    return out.reshape(b, t, _D_EMBED)
